# Initial kernel scaffold; baseline (speedup 1.0000x reference)
import jax
import jax.numpy as jnp
from jax.experimental import pallas as pl


def kernel(x, edge_index, edge_weight, H, Wz, bz, Wlz, blz, Wr, br, Wlr, blr, Wh, bh, Wlh, blh, Wl, bl):
    raise NotImplementedError("write your pallas kernel here")


# speedup vs baseline: 22.2322x; 22.2322x over previous
import dataclasses
import functools
import jax
import jax.numpy as jnp
from jax import lax
from jax.experimental import pallas as pl
from jax.experimental.pallas import tpu as pltpu
from jax.experimental.pallas import tpu_sc as plsc

NC = 2
NS = 16
NW = NC * NS
CH = 128
DEGW = 16


def _sc_compiler_params():
    cp = pltpu.CompilerParams()
    if "needs_layout_passes" in pltpu.CompilerParams.__dataclass_fields__:
        cp = dataclasses.replace(cp, needs_layout_passes=False)
    return cp


def _sc_deg(col3, ew3, ztab, zbuf, n, k):
    zr = n // NS

    @functools.partial(
        pl.kernel,
        out_type=jax.ShapeDtypeStruct((NC, n, DEGW), jnp.float32),
        mesh=plsc.VectorSubcoreMesh(core_axis_name="c", subcore_axis_name="s"),
        compiler_params=_sc_compiler_params(),
        scratch_types=[
            pltpu.VMEM((k, CH), jnp.int32),
            pltpu.VMEM((k, CH), jnp.float32),
            pltpu.VMEM((CH, DEGW), jnp.float32),
            pltpu.VMEM_SHARED((n, DEGW), jnp.float32),
        ],
    )
    def deg_kernel(col_hbm, ew_hbm, ztab_hbm, zbuf_hbm, out_hbm,
                   colv, ewv, buf, table):
        c = lax.axis_index("c")
        s = lax.axis_index("s")
        wid = c * NS + s
        pltpu.sync_copy(col_hbm.at[wid], colv)
        pltpu.sync_copy(ew_hbm.at[wid], ewv)
        pltpu.sync_copy(ztab_hbm, table.at[pl.ds(s * zr, zr)])
        pltpu.sync_copy(zbuf_hbm, buf)
        plsc.subcore_barrier()

        zero16 = jnp.zeros((16,), jnp.int32)
        iota16 = lax.iota(jnp.int32, 16)

        @pl.loop(0, k)
        def _(j):
            @pl.loop(0, CH // 16)
            def _(g):
                vals = ewv[j, pl.ds(g * 16, 16)]
                plsc.store_scatter(buf, [g * 16 + iota16, zero16], vals)
            pltpu.sync_copy(buf, table.at[colv.at[j]], add=True)

        plsc.subcore_barrier()
        pltpu.sync_copy(table.at[pl.ds(s * zr, zr)],
                        out_hbm.at[c, pl.ds(s * zr, zr)])

    return deg_kernel(col3, ew3, ztab, zbuf)


def _sc_scatter(y, row3, col3, ew3, ztab, n, k, d, dm):
    zr = n // NS

    @functools.partial(
        pl.kernel,
        out_type=jax.ShapeDtypeStruct((NC, n, d), jnp.float32),
        mesh=plsc.VectorSubcoreMesh(core_axis_name="c", subcore_axis_name="s"),
        compiler_params=_sc_compiler_params(),
        scratch_types=[
            pltpu.VMEM((k, CH), jnp.int32),
            pltpu.VMEM((k, CH), jnp.int32),
            pltpu.VMEM((k, CH), jnp.float32),
            pltpu.VMEM((CH, d), jnp.float32),
            pltpu.VMEM_SHARED((n, d), jnp.float32),
        ],
    )
    def scat_kernel(y_hbm, row_hbm, col_hbm, ew_hbm, ztab_hbm, out_hbm,
                    rowv, colv, ewv, gbuf, table):
        c = lax.axis_index("c")
        s = lax.axis_index("s")
        wid = c * NS + s
        pltpu.sync_copy(row_hbm.at[wid], rowv)
        pltpu.sync_copy(col_hbm.at[wid], colv)
        pltpu.sync_copy(ew_hbm.at[wid], ewv)
        pltpu.sync_copy(ztab_hbm, table.at[pl.ds(s * zr, zr)])
        plsc.subcore_barrier()

        @pl.loop(0, k)
        def _(j):
            pltpu.sync_copy(y_hbm.at[rowv.at[j]], gbuf)

            @pl.loop(0, CH // 16)
            def _(g):
                wv = ewv[j, pl.ds(g * 16, 16)]
                for i in range(16):
                    w = wv[i]
                    for t in range(dm // 16):
                        sl = pl.ds(t * 16, 16)
                        gbuf[g * 16 + i, sl] = gbuf[g * 16 + i, sl] * w

            pltpu.sync_copy(gbuf, table.at[colv.at[j]], add=True)

        plsc.subcore_barrier()
        pltpu.sync_copy(table.at[pl.ds(s * zr, zr)],
                        out_hbm.at[c, pl.ds(s * zr, zr)])

    return scat_kernel(y, row3, col3, ew3, ztab)


def _tc_y(x, wcat, degp, blk):
    n, f = x.shape
    d = wcat.shape[1]
    grid = n // blk

    def body(x_ref, w_ref, deg_ref, y_ref):
        deg = deg_ref[0, :, 0:1] + deg_ref[1, :, 0:1] + 1.0
        dis = lax.rsqrt(deg)
        xw = jnp.dot(x_ref[...], w_ref[...],
                     preferred_element_type=jnp.float32)
        y_ref[...] = dis * xw

    return pl.pallas_call(
        body,
        grid=(grid,),
        in_specs=[
            pl.BlockSpec((blk, f), lambda i: (i, 0)),
            pl.BlockSpec((f, d), lambda i: (0, 0)),
            pl.BlockSpec((NC, blk, DEGW), lambda i: (0, i, 0)),
        ],
        out_specs=pl.BlockSpec((blk, d), lambda i: (i, 0)),
        out_shape=jax.ShapeDtypeStruct((n, d), jnp.float32),
    )(x, wcat, degp)


def _tc_gru(sp, y, degp, h, w1, bzr, wlh, blh, wl, bl, bcat, blk):
    n, d = y.shape
    hd = h.shape[1]
    dr = 3 * hd
    c_out = wl.shape[1]
    grid = n // blk

    def body(sp_ref, y_ref, deg_ref, h_ref, w1_ref, bzr_ref, wlh_ref,
             blh_ref, wl_ref, bl_ref, bcat_ref, out_ref):
        deg = deg_ref[0, :, 0:1] + deg_ref[1, :, 0:1] + 1.0
        dis = lax.rsqrt(deg)
        a = (dis * (sp_ref[0] + sp_ref[1] + y_ref[...]))[:, :dr] \
            + bcat_ref[...]
        hb = h_ref[...]
        ah = jnp.concatenate([a, hb], axis=1)
        zr_pre = jnp.dot(ah, w1_ref[...],
                         preferred_element_type=jnp.float32) + bzr_ref[...]
        zr_act = jax.nn.sigmoid(zr_pre)
        z = zr_act[:, :hd]
        r = zr_act[:, hd:]
        ah2 = jnp.concatenate([a[:, 2 * hd:3 * hd], hb * r], axis=1)
        ht = jnp.tanh(jnp.dot(ah2, wlh_ref[...],
                              preferred_element_type=jnp.float32)
                      + blh_ref[...])
        hn = z * hb + (1.0 - z) * ht
        hr = jnp.maximum(hn, 0.0)
        logits = jnp.dot(hr, wl_ref[...],
                         preferred_element_type=jnp.float32) + bl_ref[...]
        m = jnp.max(logits, axis=1, keepdims=True)
        e = jnp.exp(logits - m)
        out_ref[...] = e / jnp.sum(e, axis=1, keepdims=True)

    return pl.pallas_call(
        body,
        grid=(grid,),
        in_specs=[
            pl.BlockSpec((NC, blk, d), lambda i: (0, i, 0)),
            pl.BlockSpec((blk, d), lambda i: (i, 0)),
            pl.BlockSpec((NC, blk, DEGW), lambda i: (0, i, 0)),
            pl.BlockSpec((blk, hd), lambda i: (i, 0)),
            pl.BlockSpec(w1.shape, lambda i: (0, 0)),
            pl.BlockSpec(bzr.shape, lambda i: (0, 0)),
            pl.BlockSpec(wlh.shape, lambda i: (0, 0)),
            pl.BlockSpec(blh.shape, lambda i: (0, 0)),
            pl.BlockSpec(wl.shape, lambda i: (0, 0)),
            pl.BlockSpec(bl.shape, lambda i: (0, 0)),
            pl.BlockSpec(bcat.shape, lambda i: (0, 0)),
        ],
        out_specs=pl.BlockSpec((blk, c_out), lambda i: (i, 0)),
        out_shape=jax.ShapeDtypeStruct((n, c_out), jnp.float32),
    )(sp, y, degp, h, w1, bzr, wlh, blh, wl, bl, bcat)


def kernel(x, edge_index, edge_weight, H, Wz, bz, Wlz, blz, Wr, br, Wlr, blr,
           Wh, bh, Wlh, blh, Wl, bl):
    n, f = x.shape
    e = edge_index.shape[1]
    hd = H.shape[1]
    d = 3 * hd

    k = -(-e // (NW * CH))
    ep = NW * k * CH
    pad = ep - e
    row = edge_index[0]
    col = edge_index[1]
    ew = edge_weight
    if pad:
        zi = jnp.zeros((pad,), edge_index.dtype)
        row = jnp.concatenate([row, zi])
        col = jnp.concatenate([col, zi])
        ew = jnp.concatenate([ew, jnp.zeros((pad,), ew.dtype)])
    row3 = row.reshape(NW, k, CH)
    col3 = col.reshape(NW, k, CH)
    ew3 = ew.reshape(NW, k, CH)

    dpad = 128
    wcat = jnp.concatenate([Wz, Wr, Wh], axis=1)
    wcat = jnp.pad(wcat, ((0, 0), (0, dpad - d)))
    w1 = jnp.zeros((d + hd, 2 * hd), jnp.float32)
    w1 = w1.at[0:hd, 0:hd].set(Wlz[:hd])
    w1 = w1.at[hd:2 * hd, hd:2 * hd].set(Wlr[:hd])
    w1 = w1.at[d:d + hd, 0:hd].set(Wlz[hd:])
    w1 = w1.at[d:d + hd, hd:2 * hd].set(Wlr[hd:])
    bzr = jnp.concatenate([blz, blr]).reshape(1, 2 * hd)
    bcat = jnp.concatenate([bz, br, bh]).reshape(1, d)
    blh2 = blh.reshape(1, hd)
    bl2 = bl.reshape(1, -1)

    n_tab = -(-n // (NS * 8)) * (NS * 8)
    zr = n_tab // NS
    ztab16 = jnp.zeros((zr, DEGW), jnp.float32)
    zbuf = jnp.zeros((CH, DEGW), jnp.float32)
    ztab128 = jnp.zeros((zr, dpad), jnp.float32)

    degp = _sc_deg(col3, ew3, ztab16, zbuf, n_tab, k)
    y = _tc_y(x, wcat, degp, blk=2000)
    sp = _sc_scatter(y, row3, col3, ew3, ztab128, n_tab, k, dpad, d)
    out = _tc_gru(sp, y, degp, H, w1, bzr, Wlh, blh2, Wl, bl2, bcat,
                  blk=2000)
    return (out, H)


# speedup vs baseline: 26.9535x; 1.2124x over previous
import dataclasses
import functools
import jax
import jax.numpy as jnp
from jax import lax
from jax.experimental import pallas as pl
from jax.experimental.pallas import tpu as pltpu
from jax.experimental.pallas import tpu_sc as plsc

NC = 2
NS = 16
NW = NC * NS
CH = 128
DEGW = 16


def _sc_compiler_params():
    cp = pltpu.CompilerParams()
    if "needs_layout_passes" in pltpu.CompilerParams.__dataclass_fields__:
        cp = dataclasses.replace(cp, needs_layout_passes=False)
    return cp


def _sc_deg(col3, ew3, zrow, n, k):
    @functools.partial(
        pl.kernel,
        out_type=jax.ShapeDtypeStruct((NW, 1, n), jnp.float32),
        mesh=plsc.VectorSubcoreMesh(core_axis_name="c", subcore_axis_name="s"),
        compiler_params=_sc_compiler_params(),
        scratch_types=[
            pltpu.VMEM((k, CH), jnp.int32),
            pltpu.VMEM((k, CH), jnp.float32),
            pltpu.VMEM((1, n), jnp.float32),
        ],
    )
    def deg_kernel(col_hbm, ew_hbm, z_hbm, out_hbm, colv, ewv, dtab):
        c = lax.axis_index("c")
        s = lax.axis_index("s")
        wid = c * NS + s
        pltpu.sync_copy(col_hbm.at[wid], colv)
        pltpu.sync_copy(ew_hbm.at[wid], ewv)
        pltpu.sync_copy(z_hbm, dtab)

        zero16 = jnp.zeros((16,), jnp.int32)

        @pl.loop(0, k)
        def _(j):
            @pl.loop(0, CH // 16)
            def _(g):
                idx = colv[j, pl.ds(g * 16, 16)]
                vals = ewv[j, pl.ds(g * 16, 16)]
                plsc.addupdate_scatter(dtab, [zero16, idx], vals)

        pltpu.sync_copy(dtab, out_hbm.at[wid])

    return deg_kernel(col3, ew3, zrow)


def _sc_scatter(y, row3, col3, ew3, ztab, n, k, d, dm):
    zr = n // NS

    @functools.partial(
        pl.kernel,
        out_type=jax.ShapeDtypeStruct((NC, n, d), jnp.float32),
        mesh=plsc.VectorSubcoreMesh(core_axis_name="c", subcore_axis_name="s"),
        compiler_params=_sc_compiler_params(),
        scratch_types=[
            pltpu.VMEM((k, CH), jnp.int32),
            pltpu.VMEM((k, CH), jnp.int32),
            pltpu.VMEM((k, CH), jnp.float32),
            pltpu.VMEM((CH, d), jnp.float32),
            pltpu.VMEM_SHARED((n, d), jnp.float32),
        ],
    )
    def scat_kernel(y_hbm, row_hbm, col_hbm, ew_hbm, ztab_hbm, out_hbm,
                    rowv, colv, ewv, gbuf, table):
        c = lax.axis_index("c")
        s = lax.axis_index("s")
        wid = c * NS + s
        pltpu.sync_copy(row_hbm.at[wid], rowv)
        pltpu.sync_copy(col_hbm.at[wid], colv)
        pltpu.sync_copy(ew_hbm.at[wid], ewv)
        pltpu.sync_copy(ztab_hbm, table.at[pl.ds(s * zr, zr)])
        plsc.subcore_barrier()

        @pl.loop(0, k)
        def _(j):
            pltpu.sync_copy(y_hbm.at[rowv.at[j]], gbuf)

            @pl.loop(0, CH // 16)
            def _(g):
                wv = ewv[j, pl.ds(g * 16, 16)]
                for i in range(16):
                    w = wv[i]
                    for t in range(dm // 16):
                        sl = pl.ds(t * 16, 16)
                        e = g * 16 + i
                        gbuf[e, sl] = gbuf[e, sl] * w

            pltpu.sync_copy(gbuf, table.at[colv.at[j]], add=True)

        plsc.subcore_barrier()
        pltpu.sync_copy(table.at[pl.ds(s * zr, zr)],
                        out_hbm.at[c, pl.ds(s * zr, zr)])

    return scat_kernel(y, row3, col3, ew3, ztab)


def _tc_y(x, wcat, degp, blk):
    n, f = x.shape
    d = wcat.shape[1]
    grid = n // blk

    def body(x_ref, w_ref, deg_ref, y_ref):
        deg = jnp.sum(deg_ref[...], axis=1, keepdims=True) + 1.0
        dis = lax.rsqrt(deg)
        xw = jnp.dot(x_ref[...], w_ref[...],
                     preferred_element_type=jnp.float32)
        y_ref[...] = dis * xw

    return pl.pallas_call(
        body,
        grid=(grid,),
        in_specs=[
            pl.BlockSpec((blk, f), lambda i: (i, 0)),
            pl.BlockSpec((f, d), lambda i: (0, 0)),
            pl.BlockSpec((blk, NW), lambda i: (i, 0)),
        ],
        out_specs=pl.BlockSpec((blk, d), lambda i: (i, 0)),
        out_shape=jax.ShapeDtypeStruct((n, d), jnp.float32),
    )(x, wcat, degp)


def _tc_gru(sp, y, degp, h, w1, bzr, wlh, blh, wl, bl, bcat, blk):
    n, d = y.shape
    hd = h.shape[1]
    dr = 3 * hd
    c_out = wl.shape[1]
    grid = n // blk

    def body(sp_ref, y_ref, deg_ref, h_ref, w1_ref, bzr_ref, wlh_ref,
             blh_ref, wl_ref, bl_ref, bcat_ref, out_ref):
        deg = jnp.sum(deg_ref[...], axis=1, keepdims=True) + 1.0
        dis = lax.rsqrt(deg)
        a = (dis * (sp_ref[0] + sp_ref[1] + y_ref[...]))[:, :dr] \
            + bcat_ref[...]
        hb = h_ref[...]
        ah = jnp.concatenate([a, hb], axis=1)
        zr_pre = jnp.dot(ah, w1_ref[...],
                         preferred_element_type=jnp.float32) + bzr_ref[...]
        zr_act = jax.nn.sigmoid(zr_pre)
        z = zr_act[:, :hd]
        r = zr_act[:, hd:]
        ah2 = jnp.concatenate([a[:, 2 * hd:3 * hd], hb * r], axis=1)
        ht = jnp.tanh(jnp.dot(ah2, wlh_ref[...],
                              preferred_element_type=jnp.float32)
                      + blh_ref[...])
        hn = z * hb + (1.0 - z) * ht
        hr = jnp.maximum(hn, 0.0)
        logits = jnp.dot(hr, wl_ref[...],
                         preferred_element_type=jnp.float32) + bl_ref[...]
        m = jnp.max(logits, axis=1, keepdims=True)
        e = jnp.exp(logits - m)
        out_ref[...] = e / jnp.sum(e, axis=1, keepdims=True)

    return pl.pallas_call(
        body,
        grid=(grid,),
        in_specs=[
            pl.BlockSpec((NC, blk, d), lambda i: (0, i, 0)),
            pl.BlockSpec((blk, d), lambda i: (i, 0)),
            pl.BlockSpec((blk, NW), lambda i: (i, 0)),
            pl.BlockSpec((blk, hd), lambda i: (i, 0)),
            pl.BlockSpec(w1.shape, lambda i: (0, 0)),
            pl.BlockSpec(bzr.shape, lambda i: (0, 0)),
            pl.BlockSpec(wlh.shape, lambda i: (0, 0)),
            pl.BlockSpec(blh.shape, lambda i: (0, 0)),
            pl.BlockSpec(wl.shape, lambda i: (0, 0)),
            pl.BlockSpec(bl.shape, lambda i: (0, 0)),
            pl.BlockSpec(bcat.shape, lambda i: (0, 0)),
        ],
        out_specs=pl.BlockSpec((blk, c_out), lambda i: (i, 0)),
        out_shape=jax.ShapeDtypeStruct((n, c_out), jnp.float32),
    )(sp, y, degp, h, w1, bzr, wlh, blh, wl, bl, bcat)


def kernel(x, edge_index, edge_weight, H, Wz, bz, Wlz, blz, Wr, br, Wlr, blr,
           Wh, bh, Wlh, blh, Wl, bl):
    n, f = x.shape
    e = edge_index.shape[1]
    hd = H.shape[1]
    d = 3 * hd

    k = -(-e // (NW * CH))
    ep = NW * k * CH
    pad = ep - e
    row = edge_index[0]
    col = edge_index[1]
    ew = edge_weight
    if pad:
        zi = jnp.zeros((pad,), edge_index.dtype)
        row = jnp.concatenate([row, zi])
        col = jnp.concatenate([col, zi])
        ew = jnp.concatenate([ew, jnp.zeros((pad,), ew.dtype)])
    row3 = row.reshape(NW, k, CH)
    col3 = col.reshape(NW, k, CH)
    ew3 = ew.reshape(NW, k, CH)

    dpad = 128
    wcat = jnp.concatenate([Wz, Wr, Wh], axis=1)
    wcat = jnp.pad(wcat, ((0, 0), (0, dpad - d)))
    w1 = jnp.zeros((d + hd, 2 * hd), jnp.float32)
    w1 = w1.at[0:hd, 0:hd].set(Wlz[:hd])
    w1 = w1.at[hd:2 * hd, hd:2 * hd].set(Wlr[:hd])
    w1 = w1.at[d:d + hd, 0:hd].set(Wlz[hd:])
    w1 = w1.at[d:d + hd, hd:2 * hd].set(Wlr[hd:])
    bzr = jnp.concatenate([blz, blr]).reshape(1, 2 * hd)
    bcat = jnp.concatenate([bz, br, bh]).reshape(1, d)
    blh2 = blh.reshape(1, hd)
    bl2 = bl.reshape(1, -1)

    n_tab = -(-n // (NS * 8)) * (NS * 8)
    zr = n_tab // NS
    ztabd = jnp.zeros((zr, dpad), jnp.float32)
    zrow = jnp.zeros((1, n_tab), jnp.float32)

    degp = _sc_deg(col3, ew3, zrow, n_tab, k)
    degt = jnp.transpose(degp.reshape(NW, n_tab))[:n]
    y = _tc_y(x, wcat, degt, blk=2000)
    sp = _sc_scatter(y, row3, col3, ew3, ztabd, n_tab, k, dpad, d)
    out = _tc_gru(sp, y, degt, H, w1, bzr, Wlh, blh2, Wl, bl2, bcat,
                  blk=2000)
    return (out, H)


# speedup vs baseline: 29.6667x; 1.1007x over previous
import dataclasses
import functools
import jax
import jax.numpy as jnp
from jax import lax
from jax.experimental import pallas as pl
from jax.experimental.pallas import tpu as pltpu
from jax.experimental.pallas import tpu_sc as plsc

NC = 2
NS = 16
NW = NC * NS
CH = 128
DEGW = 16


def _sc_compiler_params():
    cp = pltpu.CompilerParams()
    if "needs_layout_passes" in pltpu.CompilerParams.__dataclass_fields__:
        cp = dataclasses.replace(cp, needs_layout_passes=False)
    return cp


def _sc_deg(col3, ew3, zrow, n, k):
    @functools.partial(
        pl.kernel,
        out_type=jax.ShapeDtypeStruct((NW, 1, n), jnp.float32),
        mesh=plsc.VectorSubcoreMesh(core_axis_name="c", subcore_axis_name="s"),
        compiler_params=_sc_compiler_params(),
        scratch_types=[
            pltpu.VMEM((k, CH), jnp.int32),
            pltpu.VMEM((k, CH), jnp.float32),
            pltpu.VMEM((1, n), jnp.float32),
        ],
    )
    def deg_kernel(col_hbm, ew_hbm, z_hbm, out_hbm, colv, ewv, dtab):
        c = lax.axis_index("c")
        s = lax.axis_index("s")
        wid = c * NS + s
        pltpu.sync_copy(col_hbm.at[wid], colv)
        pltpu.sync_copy(ew_hbm.at[wid], ewv)
        pltpu.sync_copy(z_hbm, dtab)

        zero16 = jnp.zeros((16,), jnp.int32)

        @pl.loop(0, k)
        def _(j):
            @pl.loop(0, CH // 16)
            def _(g):
                idx = colv[j, pl.ds(g * 16, 16)]
                vals = ewv[j, pl.ds(g * 16, 16)]
                plsc.addupdate_scatter(dtab, [zero16, idx], vals)

        pltpu.sync_copy(dtab, out_hbm.at[wid])

    return deg_kernel(col3, ew3, zrow)


def _sc_scatter(y, row4, col3, ew4, ztab, n, k, d, dm):
    zr = n // NS
    assert k >= 4 and k % 2 == 1

    @functools.partial(
        pl.kernel,
        out_type=jax.ShapeDtypeStruct((NC, n, d), jnp.float32),
        mesh=plsc.VectorSubcoreMesh(core_axis_name="c", subcore_axis_name="s"),
        compiler_params=_sc_compiler_params(),
        scratch_types=[
            pltpu.VMEM((k, CH), jnp.int32),
            pltpu.VMEM((2, 1, CH), jnp.int32),
            pltpu.VMEM((2, 1, CH), jnp.float32),
            pltpu.VMEM((2, CH, d), jnp.float32),
            pltpu.VMEM_SHARED((n, d), jnp.float32),
            pltpu.SemaphoreType.DMA,
            pltpu.SemaphoreType.DMA,
            pltpu.SemaphoreType.DMA,
            pltpu.SemaphoreType.DMA,
            pltpu.SemaphoreType.DMA,
            pltpu.SemaphoreType.DMA,
        ],
    )
    def scat_kernel(y_hbm, row_hbm, col_hbm, ew_hbm, ztab_hbm, out_hbm,
                    colv, rowb, ewb, gbuf, table,
                    rsem0, rsem1, esem0, esem1, gsem0, gsem1):
        c = lax.axis_index("c")
        s = lax.axis_index("s")
        wid = c * NS + s
        rsems = (rsem0, rsem1)
        esems = (esem0, esem1)
        gsems = (gsem0, gsem1)
        pltpu.sync_copy(col_hbm.at[wid], colv)
        pltpu.sync_copy(ztab_hbm, table.at[pl.ds(s * zr, zr)])
        plsc.subcore_barrier()

        def row_copy(j, b):
            return pltpu.make_async_copy(
                row_hbm.at[wid, j], rowb.at[b], rsems[b])

        def ew_copy(j, b):
            return pltpu.make_async_copy(
                ew_hbm.at[wid, j], ewb.at[b], esems[b])

        def gather_copy(b):
            return pltpu.make_async_copy(
                y_hbm.at[rowb.at[b, 0]], gbuf.at[b], gsems[b])

        def scale(j, b):
            @pl.loop(0, CH // 16)
            def _(g):
                wv = ewb[b, 0, pl.ds(g * 16, 16)]
                for i in range(16):
                    w = wv[i]
                    for t in range(dm // 16):
                        sl = pl.ds(t * 16, 16)
                        e = g * 16 + i
                        gbuf[b, e, sl] = gbuf[b, e, sl] * w

        def step(j, b, pre_row=True, pre_next=True):
            o = 1 - b
            gather_copy(b).wait()
            if pre_row:
                row_copy(j + 2, b).start()
            if pre_next:
                row_copy(j + 1, o).wait()
                gather_copy(o).start()
                ew_copy(j + 1, o).start()
            ew_copy(j, b).wait()
            scale(j, b)
            pltpu.sync_copy(gbuf.at[b], table.at[colv.at[j]], add=True)

        row_copy(0, 0).start()
        row_copy(1, 1).start()
        ew_copy(0, 0).start()
        row_copy(0, 0).wait()
        gather_copy(0).start()
        step(0, 0)

        @pl.loop(1, k - 2, step=2)
        def _(j):
            step(j, 1)
            step(j + 1, 0)

        step(k - 2, 1, pre_row=False)
        step(k - 1, 0, pre_row=False, pre_next=False)

        plsc.subcore_barrier()
        pltpu.sync_copy(table.at[pl.ds(s * zr, zr)],
                        out_hbm.at[c, pl.ds(s * zr, zr)])

    return scat_kernel(y, row4, col3, ew4, ztab)


def _tc_y(x, wcat, degp, blk):
    n, f = x.shape
    d = wcat.shape[1]
    grid = n // blk

    def body(x_ref, w_ref, deg_ref, y_ref):
        deg = jnp.sum(deg_ref[...], axis=1, keepdims=True) + 1.0
        dis = lax.rsqrt(deg)
        xw = jnp.dot(x_ref[...], w_ref[...],
                     preferred_element_type=jnp.float32)
        y_ref[...] = dis * xw

    return pl.pallas_call(
        body,
        grid=(grid,),
        in_specs=[
            pl.BlockSpec((blk, f), lambda i: (i, 0)),
            pl.BlockSpec((f, d), lambda i: (0, 0)),
            pl.BlockSpec((blk, NW), lambda i: (i, 0)),
        ],
        out_specs=pl.BlockSpec((blk, d), lambda i: (i, 0)),
        out_shape=jax.ShapeDtypeStruct((n, d), jnp.float32),
    )(x, wcat, degp)


def _tc_gru(sp, y, degp, h, w1, bzr, wlh, blh, wl, bl, bcat, blk):
    n, d = y.shape
    hd = h.shape[1]
    dr = 3 * hd
    c_out = wl.shape[1]
    grid = n // blk

    def body(sp_ref, y_ref, deg_ref, h_ref, w1_ref, bzr_ref, wlh_ref,
             blh_ref, wl_ref, bl_ref, bcat_ref, out_ref):
        deg = jnp.sum(deg_ref[...], axis=1, keepdims=True) + 1.0
        dis = lax.rsqrt(deg)
        a = (dis * (sp_ref[0] + sp_ref[1] + y_ref[...]))[:, :dr] \
            + bcat_ref[...]
        hb = h_ref[...]
        ah = jnp.concatenate([a, hb], axis=1)
        zr_pre = jnp.dot(ah, w1_ref[...],
                         preferred_element_type=jnp.float32) + bzr_ref[...]
        zr_act = jax.nn.sigmoid(zr_pre)
        z = zr_act[:, :hd]
        r = zr_act[:, hd:]
        ah2 = jnp.concatenate([a[:, 2 * hd:3 * hd], hb * r], axis=1)
        ht = jnp.tanh(jnp.dot(ah2, wlh_ref[...],
                              preferred_element_type=jnp.float32)
                      + blh_ref[...])
        hn = z * hb + (1.0 - z) * ht
        hr = jnp.maximum(hn, 0.0)
        logits = jnp.dot(hr, wl_ref[...],
                         preferred_element_type=jnp.float32) + bl_ref[...]
        m = jnp.max(logits, axis=1, keepdims=True)
        e = jnp.exp(logits - m)
        out_ref[...] = e / jnp.sum(e, axis=1, keepdims=True)

    return pl.pallas_call(
        body,
        grid=(grid,),
        in_specs=[
            pl.BlockSpec((NC, blk, d), lambda i: (0, i, 0)),
            pl.BlockSpec((blk, d), lambda i: (i, 0)),
            pl.BlockSpec((blk, NW), lambda i: (i, 0)),
            pl.BlockSpec((blk, hd), lambda i: (i, 0)),
            pl.BlockSpec(w1.shape, lambda i: (0, 0)),
            pl.BlockSpec(bzr.shape, lambda i: (0, 0)),
            pl.BlockSpec(wlh.shape, lambda i: (0, 0)),
            pl.BlockSpec(blh.shape, lambda i: (0, 0)),
            pl.BlockSpec(wl.shape, lambda i: (0, 0)),
            pl.BlockSpec(bl.shape, lambda i: (0, 0)),
            pl.BlockSpec(bcat.shape, lambda i: (0, 0)),
        ],
        out_specs=pl.BlockSpec((blk, c_out), lambda i: (i, 0)),
        out_shape=jax.ShapeDtypeStruct((n, c_out), jnp.float32),
    )(sp, y, degp, h, w1, bzr, wlh, blh, wl, bl, bcat)


def kernel(x, edge_index, edge_weight, H, Wz, bz, Wlz, blz, Wr, br, Wlr, blr,
           Wh, bh, Wlh, blh, Wl, bl):
    n, f = x.shape
    e = edge_index.shape[1]
    hd = H.shape[1]
    d = 3 * hd

    k = -(-e // (NW * CH))
    ep = NW * k * CH
    pad = ep - e
    row = edge_index[0]
    col = edge_index[1]
    ew = edge_weight
    if pad:
        zi = jnp.zeros((pad,), edge_index.dtype)
        row = jnp.concatenate([row, zi])
        col = jnp.concatenate([col, zi])
        ew = jnp.concatenate([ew, jnp.zeros((pad,), ew.dtype)])
    row3 = row.reshape(NW, k, CH)
    col3 = col.reshape(NW, k, CH)
    ew3 = ew.reshape(NW, k, CH)

    dpad = 128
    wcat = jnp.concatenate([Wz, Wr, Wh], axis=1)
    wcat = jnp.pad(wcat, ((0, 0), (0, dpad - d)))
    w1 = jnp.zeros((d + hd, 2 * hd), jnp.float32)
    w1 = w1.at[0:hd, 0:hd].set(Wlz[:hd])
    w1 = w1.at[hd:2 * hd, hd:2 * hd].set(Wlr[:hd])
    w1 = w1.at[d:d + hd, 0:hd].set(Wlz[hd:])
    w1 = w1.at[d:d + hd, hd:2 * hd].set(Wlr[hd:])
    bzr = jnp.concatenate([blz, blr]).reshape(1, 2 * hd)
    bcat = jnp.concatenate([bz, br, bh]).reshape(1, d)
    blh2 = blh.reshape(1, hd)
    bl2 = bl.reshape(1, -1)

    n_tab = -(-n // (NS * 8)) * (NS * 8)
    zr = n_tab // NS
    ztabd = jnp.zeros((zr, dpad), jnp.float32)
    zrow = jnp.zeros((1, n_tab), jnp.float32)

    degp = _sc_deg(col3, ew3, zrow, n_tab, k)
    degt = jnp.transpose(degp.reshape(NW, n_tab))[:n]
    y = _tc_y(x, wcat, degt, blk=2000)
    sp = _sc_scatter(y, row3.reshape(NW, k, 1, CH), col3,
                     ew3.reshape(NW, k, 1, CH), ztabd, n_tab, k, dpad, d)
    out = _tc_gru(sp, y, degt, H, w1, bzr, Wlh, blh2, Wl, bl2, bcat,
                  blk=2000)
    return (out, H)
